# baseline (speedup 1.0000x reference)
import functools

import jax
import jax.numpy as jnp
from jax import lax
from jax.experimental import pallas as pl
from jax.experimental.pallas import tpu as pltpu
from jax.experimental.pallas import tpu_sc as plsc

B, C, H, W = 256, 3, 128, 128
T = 1000
L = 16
ROWS = C * H
CHW = C * H * W
R = 16

NW = 32
S_SC = 32
SPW = S_SC // NW
B_TC = B - S_SC
CH = 8192
NCH = (SPW * CHW) // CH


def _sc_fma_body(xf_hbm, nf_hbm, t_hbm, sab_hbm, somab_hbm, out_hbm,
                 t_v, tab_v, xb0, xb1, nb0, nb1, ob0, ob1,
                 sem_misc, sx0, sx1, sn0, sn1, so0, so1):
    wid = lax.axis_index("s") * 2 + lax.axis_index("c")
    base = (B_TC + wid * SPW) * CHW
    obase = wid * SPW * CHW

    cp_t = pltpu.async_copy(t_hbm.at[pl.ds(B_TC, S_SC)], t_v, sem_misc)
    cp_t1 = pltpu.async_copy(sab_hbm, tab_v.at[pl.ds(0, T)], sem_misc)
    cp_t2 = pltpu.async_copy(somab_hbm, tab_v.at[pl.ds(T, T)], sem_misc)

    xb = [xb0, xb1]
    nb = [nb0, nb1]
    ob = [ob0, ob1]
    sx = [sx0, sx1]
    sn = [sn0, sn1]
    so = [so0, so1]

    in_cps = [(
        pltpu.async_copy(xf_hbm.at[pl.ds(base, CH)], xb0, sx0),
        pltpu.async_copy(nf_hbm.at[pl.ds(base, CH)], nb0, sn0),
    )]

    cp_t.wait()
    cp_t1.wait()
    cp_t2.wait()

    out_cps = []
    for g in range(NCH):
        cur = g % 2
        smp = (g * CH) // CHW
        idxw = jnp.full((L,), wid * SPW + smp, jnp.int32)
        tsplat = plsc.load_gather(t_v, [idxw])
        s1v = plsc.load_gather(tab_v, [tsplat])
        s2v = plsc.load_gather(tab_v, [tsplat + T])

        if g >= 2:
            out_cps[g - 2].wait()
        if g + 1 < NCH:
            nxt = (g + 1) % 2
            off = base + (g + 1) * CH
            in_cps.append((
                pltpu.async_copy(xf_hbm.at[pl.ds(off, CH)], xb[nxt], sx[nxt]),
                pltpu.async_copy(nf_hbm.at[pl.ds(off, CH)], nb[nxt], sn[nxt]),
            ))
        cpx, cpn = in_cps[g]
        cpx.wait()
        cpn.wait()

        xcur, ncur, ocur = xb[cur], nb[cur], ob[cur]

        @plsc.parallel_loop(0, CH // L, unroll=8)
        def _(k):
            off = k * L
            ocur[pl.ds(off, L)] = (s1v * xcur[pl.ds(off, L)]
                                   + s2v * ncur[pl.ds(off, L)])

        out_cps.append(pltpu.async_copy(
            ob[cur], out_hbm.at[pl.ds(obase + g * CH, CH)], so[cur]))

    out_cps[NCH - 2].wait()
    out_cps[NCH - 1].wait()


def _sc_fma(xf, nf, t, sab_table, somab_table):
    mesh = plsc.VectorSubcoreMesh(core_axis_name="c", subcore_axis_name="s",
                                  num_cores=2)
    fn = functools.partial(
        pl.kernel,
        mesh=mesh,
        compiler_params=pltpu.CompilerParams(needs_layout_passes=False),
        out_type=jax.ShapeDtypeStruct((S_SC * CHW,), jnp.float32),
        scratch_types=[
            pltpu.VMEM((S_SC,), jnp.int32),
            pltpu.VMEM((2 * T,), jnp.float32),
            pltpu.VMEM((CH,), jnp.float32),
            pltpu.VMEM((CH,), jnp.float32),
            pltpu.VMEM((CH,), jnp.float32),
            pltpu.VMEM((CH,), jnp.float32),
            pltpu.VMEM((CH,), jnp.float32),
            pltpu.VMEM((CH,), jnp.float32),
            pltpu.SemaphoreType.DMA,
            pltpu.SemaphoreType.DMA,
            pltpu.SemaphoreType.DMA,
            pltpu.SemaphoreType.DMA,
            pltpu.SemaphoreType.DMA,
            pltpu.SemaphoreType.DMA,
            pltpu.SemaphoreType.DMA,
        ],
    )(_sc_fma_body)
    return fn(xf, nf, t, sab_table, somab_table)


def _tc_body(t_ref, sab_ref, somab_ref, x_ref, n_ref, o_ref):
    i = pl.program_id(0)
    for r in range(R):
        ti = t_ref[i * R + r]
        s1 = sab_ref[ti]
        s2 = somab_ref[ti]
        o_ref[r] = s1 * x_ref[r] + s2 * n_ref[r]


def _tc_fma(t, sab_table, somab_table, x3, n3):
    return pl.pallas_call(
        _tc_body,
        grid=(B_TC // R,),
        in_specs=[
            pl.BlockSpec(memory_space=pltpu.SMEM),
            pl.BlockSpec(memory_space=pltpu.SMEM),
            pl.BlockSpec(memory_space=pltpu.SMEM),
            pl.BlockSpec((R, ROWS, W), lambda i: (i, 0, 0)),
            pl.BlockSpec((R, ROWS, W), lambda i: (i, 0, 0)),
        ],
        out_specs=pl.BlockSpec((R, ROWS, W), lambda i: (i, 0, 0)),
        out_shape=jax.ShapeDtypeStruct((B_TC, ROWS, W), jnp.float32),
    )(t, sab_table, somab_table, x3, n3)


def kernel(x_0, t, noise, sqrt_alpha_bar, sqrt_one_minus_alpha_bar):
    t32 = t.astype(jnp.int32)
    xf = x_0.reshape(-1)
    nf = noise.reshape(-1)
    sc_out = _sc_fma(xf, nf, t32, sqrt_alpha_bar, sqrt_one_minus_alpha_bar)

    x3 = x_0.reshape(B, ROWS, W)
    n3 = noise.reshape(B, ROWS, W)
    tc_out = _tc_fma(t32, sqrt_alpha_bar, sqrt_one_minus_alpha_bar, x3, n3)

    x_t = jnp.concatenate(
        [tc_out.reshape(B_TC, C, H, W), sc_out.reshape(S_SC, C, H, W)], axis=0)
    return x_t, noise


# speedup vs baseline: 1.3025x; 1.3025x over previous
import functools

import jax
import jax.numpy as jnp
from jax import lax
from jax.experimental import pallas as pl
from jax.experimental.pallas import tpu as pltpu
from jax.experimental.pallas import tpu_sc as plsc

B, C, H, W = 256, 3, 128, 128
T = 1000
L = 16
ROWS = C * H
R = 16


SC_MESH_CORES = 1


def _sc_gather_body(t_hbm, sab_hbm, somab_hbm, ovals_hbm,
                    idx_v, tab_v, v_v, sem_in, sem_out):
    wid = lax.axis_index("s") * SC_MESH_CORES + lax.axis_index("c")
    nchunks = B // L

    @pl.when(wid < nchunks)
    def _():
        base = wid * L
        cp_idx = pltpu.async_copy(t_hbm.at[pl.ds(base, L)], idx_v, sem_in)
        cp_t1 = pltpu.async_copy(sab_hbm, tab_v.at[pl.ds(0, T)], sem_in)
        cp_t2 = pltpu.async_copy(somab_hbm, tab_v.at[pl.ds(T, T)], sem_in)
        cp_idx.wait()
        cp_t1.wait()
        cp_t2.wait()
        idx = idx_v[...]
        v_v[pl.ds(0, L)] = plsc.load_gather(tab_v, [idx])
        v_v[pl.ds(L, L)] = plsc.load_gather(tab_v, [idx + T])
        pltpu.async_copy(v_v, ovals_hbm.at[pl.ds(wid * 2 * L, 2 * L)],
                         sem_out).wait()


def _sc_gather(t, sab_table, somab_table):
    mesh = plsc.VectorSubcoreMesh(core_axis_name="c", subcore_axis_name="s",
                                  num_cores=SC_MESH_CORES)
    fn = functools.partial(
        pl.kernel,
        mesh=mesh,
        compiler_params=pltpu.CompilerParams(needs_layout_passes=False),
        out_type=jax.ShapeDtypeStruct((2 * B,), jnp.float32),
        scratch_types=[
            pltpu.VMEM((L,), jnp.int32),
            pltpu.VMEM((2 * T,), jnp.float32),
            pltpu.VMEM((2 * L,), jnp.float32),
            pltpu.SemaphoreType.DMA,
            pltpu.SemaphoreType.DMA,
        ],
    )(_sc_gather_body)
    return fn(t, sab_table, somab_table)


def _tc_body(vals_ref, x_ref, n_ref, o_ref):
    i = pl.program_id(0)
    for r in range(R):
        j = i * R + r
        q, l = j // L, j % L
        s1 = vals_ref[q * 2 * L + l]
        s2 = vals_ref[q * 2 * L + L + l]
        o_ref[r] = s1 * x_ref[r] + s2 * n_ref[r]


def _tc_fma(vals, x3, n3):
    return pl.pallas_call(
        _tc_body,
        grid=(B // R,),
        in_specs=[
            pl.BlockSpec(memory_space=pltpu.SMEM),
            pl.BlockSpec((R, ROWS, W), lambda i: (i, 0, 0)),
            pl.BlockSpec((R, ROWS, W), lambda i: (i, 0, 0)),
        ],
        out_specs=pl.BlockSpec((R, ROWS, W), lambda i: (i, 0, 0)),
        out_shape=jax.ShapeDtypeStruct((B, ROWS, W), jnp.float32),
    )(vals, x3, n3)


def kernel(x_0, t, noise, sqrt_alpha_bar, sqrt_one_minus_alpha_bar):
    t32 = t.astype(jnp.int32)
    vals = _sc_gather(t32, sqrt_alpha_bar, sqrt_one_minus_alpha_bar)
    x3 = x_0.reshape(B, ROWS, W)
    n3 = noise.reshape(B, ROWS, W)
    x_t = _tc_fma(vals, x3, n3)
    return x_t.reshape(B, C, H, W), noise
